# Initial kernel scaffold; baseline (speedup 1.0000x reference)
import jax
import jax.numpy as jnp
from jax.experimental import pallas as pl


def kernel(image, label, mask_location):
    raise NotImplementedError("write your pallas kernel here")


# speedup vs baseline: 1.5140x; 1.5140x over previous
import jax
import jax.numpy as jnp
from jax.experimental import pallas as pl
from jax.experimental.pallas import tpu as pltpu


def _loss_kernel(msk_ref, img_ref, lbl_ref, out_ref, acc_ref):
    i = pl.program_id(0)

    @pl.when(i == 0)
    def _init():
        acc_ref[0] = 0.0
        acc_ref[1] = 0.0

    d = img_ref[...] - lbl_ref[...]
    m = msk_ref[...]
    num = jnp.sum(jnp.where(m[:, None, :, :], d * d, 0.0))
    cnt = jnp.sum(m.astype(jnp.float32))
    acc_ref[0] += num
    acc_ref[1] += cnt

    @pl.when(i == pl.num_programs(0) - 1)
    def _fin():
        out_ref[0] = acc_ref[0] / (3.0 * acc_ref[1])


def kernel(image, label, mask_location):
    B, C, H, W = image.shape
    out = pl.pallas_call(
        _loss_kernel,
        grid=(B,),
        in_specs=[
            pl.BlockSpec((1, H, W), lambda i: (i, 0, 0)),
            pl.BlockSpec((1, C, H, W), lambda i: (i, 0, 0, 0)),
            pl.BlockSpec((1, C, H, W), lambda i: (i, 0, 0, 0)),
        ],
        out_specs=pl.BlockSpec(memory_space=pltpu.SMEM),
        out_shape=jax.ShapeDtypeStruct((1,), jnp.float32),
        scratch_shapes=[pltpu.SMEM((2,), jnp.float32)],
    )(mask_location, image, label)
    return out[0]


# speedup vs baseline: 1.5268x; 1.0084x over previous
import jax
import jax.numpy as jnp
from jax.experimental import pallas as pl
from jax.experimental.pallas import tpu as pltpu


def _loss_kernel(msk_ref, img_ref, lbl_ref, out_ref, acc_ref):
    i = pl.program_id(0)

    @pl.when(i == 0)
    def _init():
        acc_ref[0] = 0.0
        acc_ref[1] = 0.0

    d = img_ref[...] - lbl_ref[...]
    d2s = jnp.sum(d * d, axis=1)
    mf = msk_ref[...].astype(jnp.float32)
    acc_ref[0] += jnp.sum(d2s * mf)
    acc_ref[1] += jnp.sum(mf)

    @pl.when(i == pl.num_programs(0) - 1)
    def _fin():
        out_ref[0] = acc_ref[0] / (3.0 * acc_ref[1])


def kernel(image, label, mask_location):
    B, C, H, W = image.shape
    out = pl.pallas_call(
        _loss_kernel,
        grid=(B,),
        in_specs=[
            pl.BlockSpec((1, H, W), lambda i: (i, 0, 0)),
            pl.BlockSpec((1, C, H, W), lambda i: (i, 0, 0, 0)),
            pl.BlockSpec((1, C, H, W), lambda i: (i, 0, 0, 0)),
        ],
        out_specs=pl.BlockSpec(memory_space=pltpu.SMEM),
        out_shape=jax.ShapeDtypeStruct((1,), jnp.float32),
        scratch_shapes=[pltpu.SMEM((2,), jnp.float32)],
    )(mask_location, image, label)
    return out[0]
